# baseline (speedup 1.0000x reference)
import jax
import jax.numpy as jnp
from jax.experimental import pallas as pl
from jax.experimental.pallas import tpu as pltpu

N_PROXIES = 8192
N_CAMS = 8
PPC = N_PROXIES // N_CAMS
TEMP = 0.07
B = 1024
D = 256
TILE = 128
P = 2048
NT = P // TILE


def _tile_kernel(tcam_ref, feat_ref, mem_ref, src_ref, tgt_ref, w_ref, out_ref):
    t = pl.program_id(0)

    @pl.when(t == 0)
    def _init():
        out_ref[...] = jnp.zeros_like(out_ref)

    src = src_ref[0, 0, :]
    rows = jax.lax.broadcasted_iota(jnp.int32, (TILE, B), 1)
    onehot = (rows == src[:, None]).astype(jnp.float32)
    x = jax.lax.dot_general(
        onehot, feat_ref[...], (((1,), (0,)), ((), ())),
        preferred_element_type=jnp.float32)
    w = mem_ref[...]
    sim = jax.lax.dot_general(
        x, w, (((1,), (1,)), ((), ())), preferred_element_type=jnp.float32
    ) * (1.0 / TEMP)
    m = jnp.max(sim, axis=1, keepdims=True)
    lse = jnp.log(jnp.sum(jnp.exp(sim - m), axis=1)) + m[:, 0]
    tgt = tgt_ref[0, 0, :]
    cols = jax.lax.broadcasted_iota(jnp.int32, (TILE, PPC), 1)
    tlogit = jnp.sum(jnp.where(cols == tgt[:, None], sim, 0.0), axis=1)
    part = jnp.sum((lse - tlogit) * w_ref[0, 0, :])
    lane = jax.lax.broadcasted_iota(jnp.int32, (1, 128), 1)
    out_ref[...] += jnp.where(lane == 0, part, 0.0)


def kernel(batch_feat, abs_proxy_label, camid, pseudo_cluster_label, memory,
           epoch, k, inter_loss_epoch):
    camid = camid.astype(jnp.int32)
    local_tgt = (abs_proxy_label % PPC).astype(jnp.int32)

    oh = (camid[None, :] == jnp.arange(N_CAMS, dtype=jnp.int32)[:, None])
    ohi = oh.astype(jnp.int32)
    cnt = jnp.sum(ohi, axis=1)
    rank = jnp.sum(ohi * (jnp.cumsum(ohi, axis=1) - 1), axis=0)
    padded = ((cnt + TILE - 1) // TILE) * TILE
    starts = jnp.concatenate(
        [jnp.zeros((1,), jnp.int32), jnp.cumsum(padded)[:-1].astype(jnp.int32)])
    pos = starts[camid] + rank
    arangeB = jnp.arange(B, dtype=jnp.int32)
    src = jnp.full((P,), -1, jnp.int32).at[pos].set(arangeB)
    sorted_tgt = jnp.zeros((P,), jnp.int32).at[pos].set(local_tgt)
    winv = 1.0 / cnt.astype(jnp.float32)[camid]
    sorted_w = jnp.zeros((P,), jnp.float32).at[pos].set(winv)
    tile_cam = (jnp.searchsorted(
        starts, jnp.arange(NT, dtype=jnp.int32) * TILE, side="right") - 1
    ).astype(jnp.int32)

    out = pl.pallas_call(
        _tile_kernel,
        grid_spec=pltpu.PrefetchScalarGridSpec(
            num_scalar_prefetch=1,
            grid=(NT,),
            in_specs=[
                pl.BlockSpec((B, D), lambda t, tc: (0, 0)),
                pl.BlockSpec((PPC, D), lambda t, tc: (tc[t], 0)),
                pl.BlockSpec((1, 1, TILE), lambda t, tc: (t, 0, 0)),
                pl.BlockSpec((1, 1, TILE), lambda t, tc: (t, 0, 0)),
                pl.BlockSpec((1, 1, TILE), lambda t, tc: (t, 0, 0)),
            ],
            out_specs=pl.BlockSpec((1, 128), lambda t, tc: (0, 0)),
        ),
        out_shape=jax.ShapeDtypeStruct((1, 128), jnp.float32),
    )(tile_cam, batch_feat, memory,
      src.reshape(NT, 1, TILE), sorted_tgt.reshape(NT, 1, TILE),
      sorted_w.reshape(NT, 1, TILE))
    return out[0, 0]


# speedup vs baseline: 1.0283x; 1.0283x over previous
import jax
import jax.numpy as jnp
from jax.experimental import pallas as pl
from jax.experimental.pallas import tpu as pltpu

N_PROXIES = 8192
N_CAMS = 8
PPC = N_PROXIES // N_CAMS
TEMP = 0.07
B = 1024
D = 256
TILE = 128
P = 2048
NT = P // TILE


def _tile_kernel(scal_ref, feat_ref, mem_ref, src_ref, tgt_ref, w_ref, out_ref):
    feat = feat_ref[...]

    def body(t, acc):
        c = scal_ref[t]
        w = mem_ref[pl.ds(c * PPC, PPC), :]
        src = src_ref[t, 0, :]
        rows = jax.lax.broadcasted_iota(jnp.int32, (TILE, B), 1)
        onehot = (rows == src[:, None]).astype(jnp.float32)
        x = jax.lax.dot_general(
            onehot, feat, (((1,), (0,)), ((), ())),
            preferred_element_type=jnp.float32)
        sim = jax.lax.dot_general(
            x, w, (((1,), (1,)), ((), ())), preferred_element_type=jnp.float32
        ) * (1.0 / TEMP)
        m = jnp.max(sim, axis=1, keepdims=True)
        lse = jnp.log(jnp.sum(jnp.exp(sim - m), axis=1)) + m[:, 0]
        tgt = tgt_ref[t, 0, :]
        cols = jax.lax.broadcasted_iota(jnp.int32, (TILE, PPC), 1)
        tlogit = jnp.sum(jnp.where(cols == tgt[:, None], sim, 0.0), axis=1)
        return acc + jnp.sum((lse - tlogit) * w_ref[t, 0, :])

    n_real = scal_ref[NT]
    total = jax.lax.fori_loop(0, n_real, body, jnp.float32(0.0))
    lane = jax.lax.broadcasted_iota(jnp.int32, (1, 128), 1)
    out_ref[...] = jnp.where(lane == 0, total, 0.0)


def kernel(batch_feat, abs_proxy_label, camid, pseudo_cluster_label, memory,
           epoch, k, inter_loss_epoch):
    camid = camid.astype(jnp.int32)
    local_tgt = (abs_proxy_label % PPC).astype(jnp.int32)

    oh = (camid[None, :] == jnp.arange(N_CAMS, dtype=jnp.int32)[:, None])
    ohi = oh.astype(jnp.int32)
    cnt = jnp.sum(ohi, axis=1)
    rank = jnp.sum(ohi * (jnp.cumsum(ohi, axis=1) - 1), axis=0)
    padded = ((cnt + TILE - 1) // TILE) * TILE
    starts = jnp.concatenate(
        [jnp.zeros((1,), jnp.int32), jnp.cumsum(padded)[:-1].astype(jnp.int32)])
    pos = starts[camid] + rank
    arangeB = jnp.arange(B, dtype=jnp.int32)
    src = jnp.full((P,), -1, jnp.int32).at[pos].set(arangeB)
    sorted_tgt = jnp.zeros((P,), jnp.int32).at[pos].set(local_tgt)
    winv = 1.0 / cnt.astype(jnp.float32)[camid]
    sorted_w = jnp.zeros((P,), jnp.float32).at[pos].set(winv)
    tile_cam = (jnp.searchsorted(
        starts, jnp.arange(NT, dtype=jnp.int32) * TILE, side="right") - 1
    ).astype(jnp.int32)
    n_real = (starts[N_CAMS - 1] + padded[N_CAMS - 1]) // TILE
    scalars = jnp.concatenate([tile_cam, n_real[None].astype(jnp.int32)])

    out = pl.pallas_call(
        _tile_kernel,
        grid_spec=pltpu.PrefetchScalarGridSpec(
            num_scalar_prefetch=1,
            grid=(1,),
            in_specs=[
                pl.BlockSpec((B, D), lambda i, tc: (0, 0)),
                pl.BlockSpec((N_PROXIES, D), lambda i, tc: (0, 0)),
                pl.BlockSpec((NT, 1, TILE), lambda i, tc: (0, 0, 0)),
                pl.BlockSpec((NT, 1, TILE), lambda i, tc: (0, 0, 0)),
                pl.BlockSpec((NT, 1, TILE), lambda i, tc: (0, 0, 0)),
            ],
            out_specs=pl.BlockSpec((1, 128), lambda i, tc: (0, 0)),
        ),
        out_shape=jax.ShapeDtypeStruct((1, 128), jnp.float32),
    )(scalars, batch_feat, memory,
      src.reshape(NT, 1, TILE), sorted_tgt.reshape(NT, 1, TILE),
      sorted_w.reshape(NT, 1, TILE))
    return out[0, 0]


# speedup vs baseline: 2.4657x; 2.3979x over previous
import jax
import jax.numpy as jnp
from jax.experimental import pallas as pl
from jax.experimental.pallas import tpu as pltpu

N_PROXIES = 8192
N_CAMS = 8
PPC = N_PROXIES // N_CAMS
TEMP = 0.07
B = 1024
D = 256
TILE = 128
P = 2048
NT = P // TILE


def _tile_kernel(scal_ref, feat_ref, mem_ref, cam_ref, tgt_ref, out_ref):
    feat = feat_ref[...]
    camv = cam_ref[...]
    camsub = jax.lax.broadcasted_iota(jnp.int32, (N_CAMS, B), 0)
    ohi = (camsub == camv).astype(jnp.float32)
    cnt = jnp.sum(ohi, axis=1, keepdims=True)
    padded = jnp.floor((cnt + (TILE - 1)) * (1.0 / TILE)) * TILE
    r8 = jax.lax.broadcasted_iota(jnp.int32, (N_CAMS, N_CAMS), 0)
    c8 = jax.lax.broadcasted_iota(jnp.int32, (N_CAMS, N_CAMS), 1)
    strict_lt = (c8 < r8).astype(jnp.float32)
    starts = jax.lax.dot_general(
        strict_lt, padded, (((1,), (0,)), ((), ())),
        preferred_element_type=jnp.float32)
    ri = jax.lax.broadcasted_iota(jnp.int32, (B, B), 0)
    ci = jax.lax.broadcasted_iota(jnp.int32, (B, B), 1)
    lt_inc = (ri <= ci).astype(jnp.float32)
    incl = jax.lax.dot_general(
        ohi, lt_inc, (((1,), (0,)), ((), ())),
        preferred_element_type=jnp.float32)
    rank = jnp.sum(ohi * (incl - 1.0), axis=0, keepdims=True)
    pos = jnp.sum(ohi * starts, axis=0, keepdims=True) + rank
    winv = jnp.sum(jnp.where(ohi > 0, 1.0 / cnt, 0.0), axis=0,
                   keepdims=True)
    tgtf = tgt_ref[...].astype(jnp.float32)

    def body(t, acc):
        c = scal_ref[t]
        pj = (jax.lax.broadcasted_iota(jnp.int32, (TILE, 1), 0)
              + t * TILE).astype(jnp.float32)
        g = (pos == pj).astype(jnp.float32)
        x = jax.lax.dot_general(
            g, feat, (((1,), (0,)), ((), ())),
            preferred_element_type=jnp.float32)
        tgt_t = jnp.sum(g * tgtf, axis=1, keepdims=True)
        w_t = jnp.sum(g * winv, axis=1, keepdims=True)
        w = mem_ref[pl.ds(c * PPC, PPC), :]
        sim = jax.lax.dot_general(
            x, w, (((1,), (1,)), ((), ())), preferred_element_type=jnp.float32
        ) * (1.0 / TEMP)
        m = jnp.max(sim, axis=1, keepdims=True)
        lse = jnp.log(jnp.sum(jnp.exp(sim - m), axis=1, keepdims=True)) + m
        cols = jax.lax.broadcasted_iota(jnp.int32, (TILE, PPC), 1)
        tlogit = jnp.sum(jnp.where(cols == tgt_t.astype(jnp.int32), sim, 0.0),
                         axis=1, keepdims=True)
        return acc + (lse - tlogit) * w_t

    n_real = scal_ref[NT]
    acc = jax.lax.fori_loop(0, n_real, body, jnp.zeros((TILE, 1), jnp.float32))
    lane = jax.lax.broadcasted_iota(jnp.int32, (1, 128), 1)
    out_ref[...] = jnp.where(lane == 0, jnp.sum(acc), 0.0)


def kernel(batch_feat, abs_proxy_label, camid, pseudo_cluster_label, memory,
           epoch, k, inter_loss_epoch):
    camid = camid.astype(jnp.int32)
    local_tgt = (abs_proxy_label % PPC).astype(jnp.int32)

    cams = jnp.arange(N_CAMS, dtype=jnp.int32)
    cnt = jnp.sum((camid[None, :] == cams[:, None]).astype(jnp.int32), axis=1)
    padded = ((cnt + TILE - 1) // TILE) * TILE
    ends = jnp.sum(jnp.where(cams[None, :] <= cams[:, None], padded[None, :], 0),
                   axis=1)
    tile_start = jnp.arange(NT, dtype=jnp.int32) * TILE
    tile_cam = jnp.minimum(
        jnp.sum((tile_start[:, None] >= ends[None, :]).astype(jnp.int32),
                axis=1), N_CAMS - 1)
    n_real = ends[N_CAMS - 1] // TILE
    scalars = jnp.concatenate([tile_cam, n_real[None]]).astype(jnp.int32)

    out = pl.pallas_call(
        _tile_kernel,
        grid_spec=pltpu.PrefetchScalarGridSpec(
            num_scalar_prefetch=1,
            grid=(1,),
            in_specs=[
                pl.BlockSpec((B, D), lambda i, tc: (0, 0)),
                pl.BlockSpec((N_PROXIES, D), lambda i, tc: (0, 0)),
                pl.BlockSpec((1, B), lambda i, tc: (0, 0)),
                pl.BlockSpec((1, B), lambda i, tc: (0, 0)),
            ],
            out_specs=pl.BlockSpec((1, 128), lambda i, tc: (0, 0)),
        ),
        out_shape=jax.ShapeDtypeStruct((1, 128), jnp.float32),
    )(scalars, batch_feat, memory,
      camid.reshape(1, B), local_tgt.reshape(1, B))
    return out[0, 0]
